# baseline (speedup 1.0000x reference)
import dataclasses
import functools

import jax
import jax.numpy as jnp
from jax.experimental import pallas as pl
from jax.experimental.pallas import tpu as pltpu
from jax.experimental.pallas import tpu_sc as plsc

_BM = 256
_F32 = jnp.float32


def _dot3(a, w_hi, w_lo):
    a_hi = a.astype(jnp.bfloat16)
    a_lo = (a - a_hi.astype(_F32)).astype(jnp.bfloat16)
    out = jnp.dot(a_hi, w_hi, preferred_element_type=_F32)
    out += jnp.dot(a_hi, w_lo, preferred_element_type=_F32)
    out += jnp.dot(a_lo, w_hi, preferred_element_type=_F32)
    return out


def _split(hs):
    hi = hs.astype(jnp.bfloat16)
    lo = (hs - hi.astype(_F32)).astype(jnp.bfloat16)
    return hi, lo


def _prescale_body(h_ref, dc_ref, hi_ref, lo_ref):
    hs = h_ref[...] * dc_ref[...]
    hi, lo = _split(hs)
    hi_ref[...] = hi
    lo_ref[...] = lo


def _prescale(h, dcol, bn=None):
    NP, d = h.shape
    body = _prescale_body if bn is None else _bnprescale_body
    args = (h,) if bn is None else (h, bn[0].reshape(1, -1), bn[1].reshape(1, -1))
    extra_specs = [] if bn is None else [
        pl.BlockSpec((1, d), lambda k: (0, 0)),
        pl.BlockSpec((1, d), lambda k: (0, 0)),
    ]
    return pl.pallas_call(
        body,
        grid=(NP // 2048,),
        in_specs=[pl.BlockSpec((2048, d), lambda k: (k, 0))] + extra_specs
        + [pl.BlockSpec((2048, 1), lambda k: (k, 0))],
        out_specs=[
            pl.BlockSpec((2048, d), lambda k: (k, 0)),
            pl.BlockSpec((2048, d), lambda k: (k, 0)),
        ],
        out_shape=[
            jax.ShapeDtypeStruct((NP, d), jnp.bfloat16),
            jax.ShapeDtypeStruct((NP, d), jnp.bfloat16),
        ],
    )(*args, dcol)


def _bnprescale_body(z_ref, s_ref, t_ref, dc_ref, hi_ref, lo_ref):
    h = jnp.maximum(z_ref[...] * s_ref[...] + t_ref[...], 0.0)
    hs = h * dc_ref[...]
    hi, lo = _split(hs)
    hi_ref[...] = hi
    lo_ref[...] = lo


def _layer_body(m_ref, hi_ref, lo_ref, dr_ref, whi_ref, wlo_ref, b_ref,
                *out_refs, mode):
    i = pl.program_id(0)
    m = m_ref[...].astype(jnp.bfloat16)
    agg = jnp.dot(m, hi_ref[...], preferred_element_type=_F32)
    agg += jnp.dot(m, lo_ref[...], preferred_element_type=_F32)
    rows = pl.ds(i * _BM, _BM)
    agg += hi_ref[rows, :].astype(_F32) + lo_ref[rows, :].astype(_F32)
    agg *= dr_ref[...]
    z = _dot3(agg, whi_ref[...], wlo_ref[...]) + b_ref[...]
    if mode == "z":
        out_refs[0][...] = z
    else:
        hs = jnp.maximum(z, 0.0) * dr_ref[...]
        hi, lo = _split(hs)
        out_refs[0][...] = hi
        out_refs[1][...] = lo


def _layer(M, hi, lo, dinv_col, W, b, mode):
    NP = M.shape[0]
    d_in = hi.shape[1]
    d_out = W.shape[1]
    if mode == "z":
        out_specs = [pl.BlockSpec((_BM, d_out), lambda i: (i, 0))]
        out_shape = [jax.ShapeDtypeStruct((NP, d_out), _F32)]
    else:
        out_specs = [pl.BlockSpec((_BM, d_out), lambda i: (i, 0)),
                     pl.BlockSpec((_BM, d_out), lambda i: (i, 0))]
        out_shape = [jax.ShapeDtypeStruct((NP, d_out), jnp.bfloat16),
                     jax.ShapeDtypeStruct((NP, d_out), jnp.bfloat16)]
    outs = pl.pallas_call(
        functools.partial(_layer_body, mode=mode),
        grid=(NP // _BM,),
        in_specs=[
            pl.BlockSpec((_BM, NP), lambda i: (i, 0)),
            pl.BlockSpec((NP, d_in), lambda i: (0, 0)),
            pl.BlockSpec((NP, d_in), lambda i: (0, 0)),
            pl.BlockSpec((_BM, 1), lambda i: (i, 0)),
            pl.BlockSpec((d_in, d_out), lambda i: (0, 0)),
            pl.BlockSpec((d_in, d_out), lambda i: (0, 0)),
            pl.BlockSpec((1, d_out), lambda i: (0, 0)),
        ],
        out_specs=out_specs,
        out_shape=out_shape,
        compiler_params=pltpu.CompilerParams(
            dimension_semantics=("arbitrary",),
        ),
    )(M, hi, lo, dinv_col, *_wsplit(W), b.reshape(1, -1))
    return outs


def _wsplit(W):
    w_hi = W.astype(jnp.bfloat16)
    w_lo = (W - w_hi.astype(_F32)).astype(jnp.bfloat16)
    return w_hi, w_lo


def _head_body(z_ref, s_ref, t_ref, wp_ref, bp_ref, vit_ref, wlvhi_ref,
               wlvlo_ref, wlhhi_ref, wlhlo_ref, bl_ref, o_ref, acc_ref, *, nk):
    k = pl.program_id(0)

    @pl.when(k == 0)
    def _():
        acc_ref[...] = jnp.zeros_like(acc_ref)

    hb = jnp.maximum(z_ref[...] * s_ref[...] + t_ref[...], 0.0)
    hb_hi = hb.astype(jnp.bfloat16)
    hb_lo = (hb - hb_hi.astype(_F32)).astype(jnp.bfloat16)
    wp = wp_ref[...]
    wp_hi = wp.astype(jnp.bfloat16)
    wp_lo = (wp - wp_hi.astype(_F32)).astype(jnp.bfloat16)
    acc = jnp.dot(wp_hi, hb_hi, preferred_element_type=_F32)
    acc += jnp.dot(wp_hi, hb_lo, preferred_element_type=_F32)
    acc += jnp.dot(wp_lo, hb_hi, preferred_element_type=_F32)
    acc_ref[...] += acc

    @pl.when(k == nk - 1)
    def _():
        hp = acc_ref[...] + bp_ref[...]
        out = _dot3(vit_ref[...], wlvhi_ref[...], wlvlo_ref[...])
        out += _dot3(hp, wlhhi_ref[...], wlhlo_ref[...])
        o_ref[...] = out + bl_ref[...]


def _head(z4, scale, shift, Wp, bp, vit, Wlv, Wlh, blin, BK=2048):
    NP, d = z4.shape
    TP = Wp.shape[0]
    d_out = Wlh.shape[1]
    nk = NP // BK
    return pl.pallas_call(
        functools.partial(_head_body, nk=nk),
        grid=(nk,),
        in_specs=[
            pl.BlockSpec((BK, d), lambda k: (k, 0)),
            pl.BlockSpec((1, d), lambda k: (0, 0)),
            pl.BlockSpec((1, d), lambda k: (0, 0)),
            pl.BlockSpec((TP, BK), lambda k: (0, k)),
            pl.BlockSpec((TP, 1), lambda k: (0, 0)),
            pl.BlockSpec((TP, vit.shape[1]), lambda k: (0, 0)),
            pl.BlockSpec((vit.shape[1], d_out), lambda k: (0, 0)),
            pl.BlockSpec((vit.shape[1], d_out), lambda k: (0, 0)),
            pl.BlockSpec((d, d_out), lambda k: (0, 0)),
            pl.BlockSpec((d, d_out), lambda k: (0, 0)),
            pl.BlockSpec((1, d_out), lambda k: (0, 0)),
        ],
        out_specs=pl.BlockSpec((TP, d_out), lambda k: (0, 0)),
        out_shape=jax.ShapeDtypeStruct((TP, d_out), _F32),
        scratch_shapes=[pltpu.VMEM((TP, d), _F32)],
        compiler_params=pltpu.CompilerParams(
            dimension_semantics=("arbitrary",),
        ),
    )(z4, scale.reshape(1, -1), shift.reshape(1, -1), Wp, bp.reshape(-1, 1),
      vit, *_wsplit(Wlv), *_wsplit(Wlh), blin.reshape(1, -1))


def _rowsum_body(m_ref, o_ref, *, nc):
    c = pl.program_id(1)

    @pl.when(c == 0)
    def _():
        o_ref[...] = jnp.zeros_like(o_ref)

    o_ref[...] += jnp.sum(m_ref[...].astype(_F32), axis=1, keepdims=True)

    @pl.when(c == nc - 1)
    def _():
        o_ref[...] = jax.lax.rsqrt(o_ref[...] + 1.0)


def _dinv_from_M(M, BR=2048, BC=2048):
    NP = M.shape[0]
    nc = NP // BC
    return pl.pallas_call(
        functools.partial(_rowsum_body, nc=nc),
        grid=(NP // BR, nc),
        in_specs=[pl.BlockSpec((BR, BC), lambda r, c: (r, c))],
        out_specs=pl.BlockSpec((BR, 1), lambda r, c: (r, 0)),
        out_shape=jax.ShapeDtypeStruct((NP, 1), _F32),
        compiler_params=pltpu.CompilerParams(
            dimension_semantics=("arbitrary", "arbitrary"),
        ),
    )(M)


def _sc_build_adj(src, dst, NP):
    E = src.shape[0]
    NT = 16
    EC = E // NT
    HALF = NP // 2
    RB = 32
    NB = HALF // RB
    CAPL = 16
    NCH = (CAPL * 16) // 128
    CELLS = RB * NP
    ROWS = CELLS // 128
    TRW = ROWS // NT
    ZR = 32
    NZ = TRW // ZR
    ECH = 4000

    mesh = plsc.VectorSubcoreMesh(core_axis_name="c", subcore_axis_name="s")

    cp = pltpu.CompilerParams()
    if "needs_layout_passes" in pltpu.CompilerParams.__dataclass_fields__:
        cp = dataclasses.replace(cp, needs_layout_passes=False)

    @functools.partial(
        pl.kernel,
        out_type=jax.ShapeDtypeStruct((NP * NP // 128, 128), _F32),
        mesh=mesh,
        compiler_params=cp,
        scratch_types=[
            pltpu.VMEM((ECH,), jnp.int32),
            pltpu.VMEM((ECH,), jnp.int32),
            pltpu.VMEM(((NB + 1) * NCH, 1, 128), jnp.int32),
            pltpu.VMEM((NB + 1, 16), jnp.int32),
            pltpu.VMEM((ZR, 128), _F32),
            pltpu.VMEM((128, 128), _F32),
            pltpu.VMEM((128,), jnp.int32),
            pltpu.VMEM_SHARED((ROWS + 1, 128), _F32),
        ],
    )
    def build(src_hbm, dst_hbm, z_hbm, m_hbm, dstc, srcc, bkt, cnt,
              zeros, vals, idxhi, acc):
        core = jax.lax.axis_index("c")
        t = jax.lax.axis_index("s")
        dump = jnp.int32(CELLS)

        pltpu.sync_copy(z_hbm, zeros)

        z16 = jnp.zeros((16,), _F32)
        o16 = jnp.ones((16,), _F32)
        d16 = jnp.full((16,), dump, jnp.int32)
        i16 = jax.lax.iota(jnp.int32, 16)

        @pl.loop(0, 128)
        def _(r):
            @pl.loop(0, 128, step=16)
            def _(l):
                vals[r, pl.ds(l, 16)] = z16

        @pl.loop(0, (NB + 1) * NCH)
        def _(c):
            @pl.loop(0, 128, step=16)
            def _(l):
                bkt[c, 0, pl.ds(l, 16)] = d16

        @pl.loop(0, NB + 1)
        def _(b):
            cnt[b, :] = jnp.zeros((16,), jnp.int32)

        base_row = core * HALF

        @pl.loop(0, EC, step=ECH)
        def _(c0):
            pltpu.sync_copy(dst_hbm.at[pl.ds(t * EC + c0, ECH)], dstc)
            pltpu.sync_copy(src_hbm.at[pl.ds(t * EC + c0, ECH)], srcc)

            @pl.loop(0, ECH, step=16)
            def _(i):
                d = dstc[pl.ds(i, 16)]
                sv = srcc[pl.ds(i, 16)]
                rel = d - base_row
                b = jax.lax.shift_right_arithmetic(rel, 5)
                ok = jnp.logical_and(b >= 0, b < NB)
                bi = jnp.where(ok, b, jnp.int32(NB))
                n = plsc.load_gather(cnt, [bi, i16])
                slot = jax.lax.rem(n, jnp.int32(CAPL))
                cell = jnp.where(ok,
                                 jnp.bitwise_and(rel, RB - 1) * NP + sv, dump)
                row = bi * NCH + jax.lax.shift_right_arithmetic(slot, 3)
                col = jnp.bitwise_and(slot, 7) * 16 + i16
                plsc.store_scatter(bkt,
                                   [row, jnp.zeros((16,), jnp.int32), col],
                                   cell)
                plsc.store_scatter(cnt, [bi, i16], n + 1)

        plsc.subcore_barrier()

        @pl.loop(0, NB)
        def _(b):
            @pl.loop(0, NZ)
            def _(z):
                pltpu.sync_copy(zeros, acc.at[pl.ds(t * TRW + z * ZR, ZR), :])

            plsc.subcore_barrier()
            nch = jax.lax.shift_right_arithmetic(jnp.max(cnt[b, :]) + 7, 3)
            for j in range(NCH):
                @pl.when(j < nch)
                def _():
                    for g in range(8):
                        ids = bkt[b * NCH + j, 0, pl.ds(g * 16, 16)]
                        hi = jax.lax.shift_right_arithmetic(ids, 7)
                        low = jnp.bitwise_and(ids, 127)
                        idxhi[pl.ds(g * 16, 16)] = hi
                        plsc.store_scatter(vals, [i16 + g * 16, low], o16)
                    pltpu.sync_copy(vals, acc.at[idxhi], add=True)
                    for g in range(8):
                        ids = bkt[b * NCH + j, 0, pl.ds(g * 16, 16)]
                        low = jnp.bitwise_and(ids, 127)
                        plsc.store_scatter(vals, [i16 + g * 16, low], z16)
            plsc.subcore_barrier()
            out0 = (core * NB + b) * ROWS + t * TRW
            pltpu.sync_copy(acc.at[pl.ds(t * TRW, TRW), :],
                            m_hbm.at[pl.ds(out0, TRW), :])

    zc = jnp.zeros((ZR, 128), _F32)
    return build(src, dst, zc)


def _bn_coeffs(z, N, g, be):
    zn = z[:N]
    m = zn.mean(axis=0)
    v = zn.var(axis=0)
    scale = g * jax.lax.rsqrt(v + 1e-5)
    shift = be - m * scale
    return scale, shift


def kernel(features_list, edges_list, vit_output, W1, b1, W2, b2, g1, be1,
           W3, b3, W4, b4, g2, be2, Wproj, bproj, Wlin, blin):
    x = features_list[0]
    ei = edges_list[0]
    vit = vit_output[0]
    N = x.shape[0]
    NP = ((N + 2047) // 2048) * 2048
    T = Wproj.shape[0]
    TP = ((T + 255) // 256) * 256

    src = ei[0]
    dst = ei[1]
    M = _sc_build_adj(src, dst, NP).reshape(NP, NP)

    dinvp = _dinv_from_M(M)

    xp = jnp.zeros((NP, x.shape[1]), _F32).at[:N].set(x)
    hi0, lo0 = _prescale(xp, dinvp)

    hi1, lo1 = _layer(M, hi0, lo0, dinvp, W1, b1, mode="hs")
    z2, = _layer(M, hi1, lo1, dinvp, W2, b2, mode="z")
    s1, t1 = _bn_coeffs(z2, N, g1, be1)
    hi2, lo2 = _prescale(z2, dinvp, bn=(s1, t1))
    hi3, lo3 = _layer(M, hi2, lo2, dinvp, W3, b3, mode="hs")
    z4, = _layer(M, hi3, lo3, dinvp, W4, b4, mode="z")
    s2, t2 = _bn_coeffs(z4, N, g2, be2)

    Wp = jnp.zeros((TP, NP), _F32).at[:T, :N].set(Wproj)
    bp = jnp.zeros((TP,), _F32).at[:T].set(bproj)
    vitp = jnp.zeros((TP, vit.shape[1]), _F32).at[:T].set(vit)
    Wlv = Wlin[: vit.shape[1]]
    Wlh = Wlin[vit.shape[1]:]

    out = _head(z4, s2, t2, Wp, bp, vitp, Wlv, Wlh, blin)
    return out[:T][None]


# speedup vs baseline: 1.1676x; 1.1676x over previous
import dataclasses
import functools

import jax
import jax.numpy as jnp
from jax.experimental import pallas as pl
from jax.experimental.pallas import tpu as pltpu
from jax.experimental.pallas import tpu_sc as plsc

_BM = 256
_F32 = jnp.float32


def _dot3(a, w_hi, w_lo):
    a_hi = a.astype(jnp.bfloat16)
    a_lo = (a - a_hi.astype(_F32)).astype(jnp.bfloat16)
    out = jnp.dot(a_hi, w_hi, preferred_element_type=_F32)
    out += jnp.dot(a_hi, w_lo, preferred_element_type=_F32)
    out += jnp.dot(a_lo, w_hi, preferred_element_type=_F32)
    return out


def _split(hs):
    hi = hs.astype(jnp.bfloat16)
    lo = (hs - hi.astype(_F32)).astype(jnp.bfloat16)
    return hi, lo


def _prescale_body(h_ref, dc_ref, hi_ref, lo_ref):
    hs = h_ref[...] * dc_ref[...]
    hi, lo = _split(hs)
    hi_ref[...] = hi
    lo_ref[...] = lo


def _prescale(h, dcol, bn=None):
    NP, d = h.shape
    body = _prescale_body if bn is None else _bnprescale_body
    args = (h,) if bn is None else (h, bn[0].reshape(1, -1), bn[1].reshape(1, -1))
    extra_specs = [] if bn is None else [
        pl.BlockSpec((1, d), lambda k: (0, 0)),
        pl.BlockSpec((1, d), lambda k: (0, 0)),
    ]
    return pl.pallas_call(
        body,
        grid=(NP // 2048,),
        in_specs=[pl.BlockSpec((2048, d), lambda k: (k, 0))] + extra_specs
        + [pl.BlockSpec((2048, 1), lambda k: (k, 0))],
        out_specs=[
            pl.BlockSpec((2048, d), lambda k: (k, 0)),
            pl.BlockSpec((2048, d), lambda k: (k, 0)),
        ],
        out_shape=[
            jax.ShapeDtypeStruct((NP, d), jnp.bfloat16),
            jax.ShapeDtypeStruct((NP, d), jnp.bfloat16),
        ],
    )(*args, dcol)


def _bnprescale_body(z_ref, s_ref, t_ref, dc_ref, hi_ref, lo_ref):
    h = jnp.maximum(z_ref[...] * s_ref[...] + t_ref[...], 0.0)
    hs = h * dc_ref[...]
    hi, lo = _split(hs)
    hi_ref[...] = hi
    lo_ref[...] = lo


def _layer_body(m_ref, hi_ref, lo_ref, dr_ref, whi_ref, wlo_ref, b_ref,
                *out_refs, mode):
    i = pl.program_id(0)
    m = m_ref[...].astype(jnp.bfloat16)
    agg = jnp.dot(m, hi_ref[...], preferred_element_type=_F32)
    agg += jnp.dot(m, lo_ref[...], preferred_element_type=_F32)
    rows = pl.ds(i * _BM, _BM)
    agg += hi_ref[rows, :].astype(_F32) + lo_ref[rows, :].astype(_F32)
    agg *= dr_ref[...]
    z = _dot3(agg, whi_ref[...], wlo_ref[...]) + b_ref[...]
    if mode == "z":
        out_refs[0][...] = z
    else:
        hs = jnp.maximum(z, 0.0) * dr_ref[...]
        hi, lo = _split(hs)
        out_refs[0][...] = hi
        out_refs[1][...] = lo


def _layer(M, hi, lo, dinv_col, W, b, mode):
    NP = M.shape[0]
    d_in = hi.shape[1]
    d_out = W.shape[1]
    if mode == "z":
        out_specs = [pl.BlockSpec((_BM, d_out), lambda i: (i, 0))]
        out_shape = [jax.ShapeDtypeStruct((NP, d_out), _F32)]
    else:
        out_specs = [pl.BlockSpec((_BM, d_out), lambda i: (i, 0)),
                     pl.BlockSpec((_BM, d_out), lambda i: (i, 0))]
        out_shape = [jax.ShapeDtypeStruct((NP, d_out), jnp.bfloat16),
                     jax.ShapeDtypeStruct((NP, d_out), jnp.bfloat16)]
    outs = pl.pallas_call(
        functools.partial(_layer_body, mode=mode),
        grid=(NP // _BM,),
        in_specs=[
            pl.BlockSpec((_BM, NP), lambda i: (i, 0)),
            pl.BlockSpec((NP, d_in), lambda i: (0, 0)),
            pl.BlockSpec((NP, d_in), lambda i: (0, 0)),
            pl.BlockSpec((_BM, 1), lambda i: (i, 0)),
            pl.BlockSpec((d_in, d_out), lambda i: (0, 0)),
            pl.BlockSpec((d_in, d_out), lambda i: (0, 0)),
            pl.BlockSpec((1, d_out), lambda i: (0, 0)),
        ],
        out_specs=out_specs,
        out_shape=out_shape,
        compiler_params=pltpu.CompilerParams(
            dimension_semantics=("arbitrary",),
        ),
    )(M, hi, lo, dinv_col, *_wsplit(W), b.reshape(1, -1))
    return outs


def _wsplit(W):
    w_hi = W.astype(jnp.bfloat16)
    w_lo = (W - w_hi.astype(_F32)).astype(jnp.bfloat16)
    return w_hi, w_lo


def _head_body(z_ref, s_ref, t_ref, wp_ref, bp_ref, vit_ref, wlvhi_ref,
               wlvlo_ref, wlhhi_ref, wlhlo_ref, bl_ref, o_ref, acc_ref, *, nk):
    k = pl.program_id(0)

    @pl.when(k == 0)
    def _():
        acc_ref[...] = jnp.zeros_like(acc_ref)

    hb = jnp.maximum(z_ref[...] * s_ref[...] + t_ref[...], 0.0)
    hb_hi = hb.astype(jnp.bfloat16)
    hb_lo = (hb - hb_hi.astype(_F32)).astype(jnp.bfloat16)
    wp = wp_ref[...]
    wp_hi = wp.astype(jnp.bfloat16)
    wp_lo = (wp - wp_hi.astype(_F32)).astype(jnp.bfloat16)
    acc = jnp.dot(wp_hi, hb_hi, preferred_element_type=_F32)
    acc += jnp.dot(wp_hi, hb_lo, preferred_element_type=_F32)
    acc += jnp.dot(wp_lo, hb_hi, preferred_element_type=_F32)
    acc_ref[...] += acc

    @pl.when(k == nk - 1)
    def _():
        hp = acc_ref[...] + bp_ref[...]
        out = _dot3(vit_ref[...], wlvhi_ref[...], wlvlo_ref[...])
        out += _dot3(hp, wlhhi_ref[...], wlhlo_ref[...])
        o_ref[...] = out + bl_ref[...]


def _head(z4, scale, shift, Wp, bp, vit, Wlv, Wlh, blin, BK=2048):
    NP, d = z4.shape
    TP = Wp.shape[0]
    d_out = Wlh.shape[1]
    nk = NP // BK
    return pl.pallas_call(
        functools.partial(_head_body, nk=nk),
        grid=(nk,),
        in_specs=[
            pl.BlockSpec((BK, d), lambda k: (k, 0)),
            pl.BlockSpec((1, d), lambda k: (0, 0)),
            pl.BlockSpec((1, d), lambda k: (0, 0)),
            pl.BlockSpec((TP, BK), lambda k: (0, k)),
            pl.BlockSpec((TP, 1), lambda k: (0, 0)),
            pl.BlockSpec((TP, vit.shape[1]), lambda k: (0, 0)),
            pl.BlockSpec((vit.shape[1], d_out), lambda k: (0, 0)),
            pl.BlockSpec((vit.shape[1], d_out), lambda k: (0, 0)),
            pl.BlockSpec((d, d_out), lambda k: (0, 0)),
            pl.BlockSpec((d, d_out), lambda k: (0, 0)),
            pl.BlockSpec((1, d_out), lambda k: (0, 0)),
        ],
        out_specs=pl.BlockSpec((TP, d_out), lambda k: (0, 0)),
        out_shape=jax.ShapeDtypeStruct((TP, d_out), _F32),
        scratch_shapes=[pltpu.VMEM((TP, d), _F32)],
        compiler_params=pltpu.CompilerParams(
            dimension_semantics=("arbitrary",),
        ),
    )(z4, scale.reshape(1, -1), shift.reshape(1, -1), Wp, bp.reshape(-1, 1),
      vit, *_wsplit(Wlv), *_wsplit(Wlh), blin.reshape(1, -1))


def _rowsum_body(m_ref, o_ref, *, nc):
    c = pl.program_id(1)

    @pl.when(c == 0)
    def _():
        o_ref[...] = jnp.zeros_like(o_ref)

    o_ref[...] += jnp.sum(m_ref[...].astype(_F32), axis=1, keepdims=True)

    @pl.when(c == nc - 1)
    def _():
        o_ref[...] = jax.lax.rsqrt(o_ref[...] + 1.0)


def _dinv_from_M(M, BR=2048, BC=2048):
    NP = M.shape[0]
    nc = NP // BC
    return pl.pallas_call(
        functools.partial(_rowsum_body, nc=nc),
        grid=(NP // BR, nc),
        in_specs=[pl.BlockSpec((BR, BC), lambda r, c: (r, c))],
        out_specs=pl.BlockSpec((BR, 1), lambda r, c: (r, 0)),
        out_shape=jax.ShapeDtypeStruct((NP, 1), _F32),
        compiler_params=pltpu.CompilerParams(
            dimension_semantics=("arbitrary", "arbitrary"),
        ),
    )(M)


def _sc_build_adj(src, dst, NP):
    E = src.shape[0]
    NT = 16
    EC = E // NT
    HALF = NP // 2
    RB = 32
    NB = HALF // RB
    CAPL = 16
    NCH = (CAPL * 16) // 128
    CELLS = RB * NP
    ROWS = CELLS // 128
    TRW = ROWS // NT
    ZR = 32
    NZ = TRW // ZR
    ECH = 2000

    mesh = plsc.VectorSubcoreMesh(core_axis_name="c", subcore_axis_name="s")

    cp = pltpu.CompilerParams()
    if "needs_layout_passes" in pltpu.CompilerParams.__dataclass_fields__:
        cp = dataclasses.replace(cp, needs_layout_passes=False)

    @functools.partial(
        pl.kernel,
        out_type=jax.ShapeDtypeStruct((NP * NP // 128, 128), _F32),
        mesh=mesh,
        compiler_params=cp,
        scratch_types=[
            pltpu.VMEM((ECH,), jnp.int32),
            pltpu.VMEM((ECH,), jnp.int32),
            pltpu.VMEM(((NB + 1) * NCH, 1, 128), jnp.int32),
            pltpu.VMEM((NB + 1, 16), jnp.int32),
            pltpu.VMEM((ZR, 128), _F32),
            pltpu.VMEM((128, 128), _F32),
            pltpu.VMEM((128,), jnp.int32),
            pltpu.VMEM_SHARED((ROWS + 1, 128), _F32),
            pltpu.VMEM_SHARED((ROWS + 1, 128), _F32),
            pltpu.SemaphoreType.DMA,
            pltpu.SemaphoreType.DMA,
            pltpu.SemaphoreType.DMA,
        ],
    )
    def build(src_hbm, dst_hbm, z_hbm, m_hbm, dstc, srcc, bkt, cnt,
              zeros, vals, idxhi, acc0, acc1, zsem, dsem0, dsem1):
        core = jax.lax.axis_index("c")
        t = jax.lax.axis_index("s")
        dump = jnp.int32(CELLS)

        pltpu.sync_copy(z_hbm, zeros)

        z16 = jnp.zeros((16,), _F32)
        o16 = jnp.ones((16,), _F32)
        d16 = jnp.full((16,), dump, jnp.int32)
        i16 = jax.lax.iota(jnp.int32, 16)

        @pl.loop(0, 128)
        def _(r):
            @pl.loop(0, 128, step=16)
            def _(l):
                vals[r, pl.ds(l, 16)] = z16

        @pl.loop(0, (NB + 1) * NCH)
        def _(c):
            @pl.loop(0, 128, step=16)
            def _(l):
                bkt[c, 0, pl.ds(l, 16)] = d16

        @pl.loop(0, NB + 1)
        def _(b):
            cnt[b, :] = jnp.zeros((16,), jnp.int32)

        base_row = core * HALF

        @pl.loop(0, EC, step=ECH)
        def _(c0):
            pltpu.sync_copy(dst_hbm.at[pl.ds(t * EC + c0, ECH)], dstc)
            pltpu.sync_copy(src_hbm.at[pl.ds(t * EC + c0, ECH)], srcc)

            @pl.loop(0, ECH, step=16)
            def _(i):
                d = dstc[pl.ds(i, 16)]
                sv = srcc[pl.ds(i, 16)]
                rel = d - base_row
                b = jax.lax.shift_right_arithmetic(rel, 5)
                ok = jnp.logical_and(b >= 0, b < NB)
                bi = jnp.where(ok, b, jnp.int32(NB))
                n = plsc.load_gather(cnt, [bi, i16])
                slot = jax.lax.rem(n, jnp.int32(CAPL))
                cell = jnp.where(ok,
                                 jnp.bitwise_and(rel, RB - 1) * NP + sv, dump)
                row = bi * NCH + jax.lax.shift_right_arithmetic(slot, 3)
                col = jnp.bitwise_and(slot, 7) * 16 + i16
                plsc.store_scatter(bkt,
                                   [row, jnp.zeros((16,), jnp.int32), col],
                                   cell)
                plsc.store_scatter(cnt, [bi, i16], n + 1)

        plsc.subcore_barrier()

        @pl.loop(0, NB, step=2)
        def _(b0):
            for p in range(2):
                b = b0 + p
                acc = acc0 if p == 0 else acc1
                dsem = dsem0 if p == 0 else dsem1

                @pl.when(b0 >= 2)
                def _():
                    pltpu.make_async_copy(
                        acc.at[pl.ds(t * TRW, TRW), :],
                        m_hbm.at[pl.ds(t * TRW, TRW), :], dsem).wait()

                zcps = [
                    pltpu.async_copy(
                        zeros, acc.at[pl.ds(t * TRW + z * ZR, ZR), :], zsem)
                    for z in range(NZ)
                ]
                for c in zcps:
                    c.wait()
                plsc.subcore_barrier()
                nch = jax.lax.shift_right_arithmetic(jnp.max(cnt[b, :]) + 7, 3)
                for j in range(NCH):
                    @pl.when(j < nch)
                    def _():
                        for g in range(8):
                            ids = bkt[b * NCH + j, 0, pl.ds(g * 16, 16)]
                            hi = jax.lax.shift_right_arithmetic(ids, 7)
                            low = jnp.bitwise_and(ids, 127)
                            idxhi[pl.ds(g * 16, 16)] = hi
                            plsc.store_scatter(vals, [i16 + g * 16, low], o16)
                        pltpu.sync_copy(vals, acc.at[idxhi], add=True)
                        for g in range(8):
                            ids = bkt[b * NCH + j, 0, pl.ds(g * 16, 16)]
                            low = jnp.bitwise_and(ids, 127)
                            plsc.store_scatter(vals, [i16 + g * 16, low], z16)
                plsc.subcore_barrier()
                out0 = (core * NB + b) * ROWS + t * TRW
                pltpu.async_copy(acc.at[pl.ds(t * TRW, TRW), :],
                                 m_hbm.at[pl.ds(out0, TRW), :], dsem)

        for p in range(2):
            acc = acc0 if p == 0 else acc1
            dsem = dsem0 if p == 0 else dsem1
            pltpu.make_async_copy(acc.at[pl.ds(t * TRW, TRW), :],
                                  m_hbm.at[pl.ds(t * TRW, TRW), :],
                                  dsem).wait()

    zc = jnp.zeros((ZR, 128), _F32)
    return build(src, dst, zc)


def _bn_coeffs(z, N, g, be):
    zn = z[:N]
    m = zn.mean(axis=0)
    v = zn.var(axis=0)
    scale = g * jax.lax.rsqrt(v + 1e-5)
    shift = be - m * scale
    return scale, shift


def kernel(features_list, edges_list, vit_output, W1, b1, W2, b2, g1, be1,
           W3, b3, W4, b4, g2, be2, Wproj, bproj, Wlin, blin):
    x = features_list[0]
    ei = edges_list[0]
    vit = vit_output[0]
    N = x.shape[0]
    NP = ((N + 2047) // 2048) * 2048
    T = Wproj.shape[0]
    TP = ((T + 255) // 256) * 256

    src = ei[0]
    dst = ei[1]
    M = _sc_build_adj(src, dst, NP).reshape(NP, NP)

    dinvp = _dinv_from_M(M)

    xp = jnp.zeros((NP, x.shape[1]), _F32).at[:N].set(x)
    hi0, lo0 = _prescale(xp, dinvp)

    hi1, lo1 = _layer(M, hi0, lo0, dinvp, W1, b1, mode="hs")
    z2, = _layer(M, hi1, lo1, dinvp, W2, b2, mode="z")
    s1, t1 = _bn_coeffs(z2, N, g1, be1)
    hi2, lo2 = _prescale(z2, dinvp, bn=(s1, t1))
    hi3, lo3 = _layer(M, hi2, lo2, dinvp, W3, b3, mode="hs")
    z4, = _layer(M, hi3, lo3, dinvp, W4, b4, mode="z")
    s2, t2 = _bn_coeffs(z4, N, g2, be2)

    Wp = jnp.zeros((TP, NP), _F32).at[:T, :N].set(Wproj)
    bp = jnp.zeros((TP,), _F32).at[:T].set(bproj)
    vitp = jnp.zeros((TP, vit.shape[1]), _F32).at[:T].set(vit)
    Wlv = Wlin[: vit.shape[1]]
    Wlh = Wlin[vit.shape[1]:]

    out = _head(z4, s2, t2, Wp, bp, vitp, Wlv, Wlh, blin)
    return out[:T][None]
